# baseline (speedup 1.0000x reference)
import jax
import jax.numpy as jnp
from jax import lax
from jax.experimental import pallas as pl
from jax.experimental.pallas import tpu as pltpu
from jax.experimental.pallas import tpu_sc as plsc

VOCAB = 100000
MAX_SEQ_LEN = 512
DIM = 128
B, L = 4096, 200

_INFO = plsc.get_sparse_core_info()
_NC, _NS = _INFO.num_cores, _INFO.num_subcores
_NW = _NC * _NS

_N_ROWS = B * L
_ROWS_PER_W = _N_ROWS // _NW
_CHUNK = 128
_N_CHUNKS = _ROWS_PER_W // _CHUNK
_NBUF = 4
_N_GROUPS = _N_CHUNKS // _NBUF
_PDIM = DIM // 2


def _gather_kernel(table_hbm, ids_hbm, out_hbm, idx_v, ibuf, fbuf, *sems):
    wid = lax.axis_index("s") * _NC + lax.axis_index("c")
    row0 = wid * _ROWS_PER_W
    chunk0 = wid * _N_CHUNKS

    pltpu.sync_copy(ids_hbm.at[pl.ds(chunk0, _N_CHUNKS)], idx_v)

    gin = sems[:_NBUF]
    gout = sems[_NBUF:]

    def start(j, b):
        pltpu.async_copy(table_hbm.at[idx_v.at[j]], ibuf.at[b], gin[b])

    def wait(j, b):
        pltpu.make_async_copy(table_hbm.at[idx_v.at[j]], ibuf.at[b],
                              gin[b]).wait()

    def start_out(j, b):
        pltpu.async_copy(ibuf.at[b],
                         out_hbm.at[pl.ds(row0 + j * _CHUNK, _CHUNK)],
                         gout[b])

    def wait_out(j, b):
        pltpu.make_async_copy(ibuf.at[b],
                              out_hbm.at[pl.ds(row0 + j * _CHUNK, _CHUNK)],
                              gout[b]).wait()

    def unpack(b):
        hi_mask = jnp.int32(-65536)

        def row(r, carry):
            for k in range(4):
                w = ibuf[b, r, pl.ds(16 * k, 16)]
                fbuf[b, r, pl.ds(32 * k, 16)] = w << 16
                fbuf[b, r, pl.ds(32 * k + 16, 16)] = w & hi_mask
            return carry

        lax.fori_loop(0, _CHUNK, row, 0)

    for b in range(_NBUF):
        start(b, b)

    def body(g, carry):
        for b in range(_NBUF):
            j = g * _NBUF + b
            wait(j, b)

            @pl.when(g > 0)
            def _():
                wait_out(j - _NBUF, b)

            jn = j + _NBUF

            @pl.when(jn < _N_CHUNKS)
            def _():
                start(jn, b)

            start_out(j, b)
        return carry

    lax.fori_loop(0, _N_GROUPS, body, 0)

    for b in range(_NBUF):
        wait_out(_N_CHUNKS - _NBUF + b, b)


def _token_gather(ids_2d, table_packed):
    mesh = plsc.VectorSubcoreMesh(core_axis_name="c", subcore_axis_name="s")
    return pl.kernel(
        _gather_kernel,
        mesh=mesh,
        out_type=jax.ShapeDtypeStruct((_N_ROWS, _PDIM), jnp.int32),
        compiler_params=pltpu.CompilerParams(use_tc_tiling_on_sc=False),
        scratch_types=[
            pltpu.VMEM((_N_CHUNKS, _CHUNK), jnp.int32),
            pltpu.VMEM((_NBUF, _CHUNK, _PDIM), jnp.int32),
            pltpu.VMEM((_NBUF, _CHUNK, DIM), jnp.int32),
        ] + [pltpu.SemaphoreType.DMA] * (2 * _NBUF),
    )(table_packed, ids_2d)


_POS_BLK = 16


def _pos_kernel(pos_ref, out_ref):
    out_ref[...] = jnp.broadcast_to(pos_ref[...][None], out_ref.shape)


def _pos_broadcast(W_pos_l):
    return pl.pallas_call(
        _pos_kernel,
        grid=(B // _POS_BLK,),
        in_specs=[pl.BlockSpec((L, DIM), lambda i: (0, 0))],
        out_specs=pl.BlockSpec((_POS_BLK, L, DIM), lambda i: (i, 0, 0)),
        out_shape=jax.ShapeDtypeStruct((B, L, DIM), jnp.float32),
    )(W_pos_l)


def kernel(input_ids, W_word, W_pos):
    ids_2d = input_ids.astype(jnp.int32).reshape(_N_ROWS // _CHUNK, _CHUNK)
    W16 = W_word.astype(jnp.bfloat16).reshape(VOCAB, 4, 2, 16)
    table_packed = jax.lax.bitcast_convert_type(
        W16.transpose(0, 1, 3, 2), jnp.int32).reshape(VOCAB, _PDIM)
    X_token = jax.lax.bitcast_convert_type(
        _token_gather(ids_2d, table_packed), jnp.float32).reshape(B, L, _PDIM)
    X_pos = _pos_broadcast(W_pos[:L])
    return (X_token, X_pos)


# speedup vs baseline: 1.1928x; 1.1928x over previous
import functools

import jax
import jax.numpy as jnp
from jax import lax
from jax.experimental import pallas as pl
from jax.experimental.pallas import tpu as pltpu
from jax.experimental.pallas import tpu_sc as plsc

VOCAB = 100000
MAX_SEQ_LEN = 512
DIM = 128
B, L = 4096, 200

_INFO = plsc.get_sparse_core_info()
_NC, _NS = _INFO.num_cores, _INFO.num_subcores
_NW = _NC * _NS

_N_ROWS = B * L
_NSLAB = 5
_SLAB_ROWS = _N_ROWS // _NSLAB
_ROWS_PER_W = _SLAB_ROWS // _NW
_CHUNK = 128
_N_CHUNKS = _ROWS_PER_W // _CHUNK
_NBUF = 5
_N_GROUPS = _N_CHUNKS // _NBUF
_QCHUNK = _CHUNK // 2


def _gather_pack_kernel(table_hbm, ids_hbm, scr_hbm, idx_v, ibuf, obuf, *sems):
    wid = lax.axis_index("s") * _NC + lax.axis_index("c")
    qrow0 = wid * (_ROWS_PER_W // 2)
    chunk0 = wid * _N_CHUNKS

    pltpu.sync_copy(ids_hbm.at[pl.ds(chunk0, _N_CHUNKS)], idx_v)

    gin = sems[:_NBUF]
    gout = sems[_NBUF:]

    def start(j, b):
        pltpu.async_copy(table_hbm.at[idx_v.at[j]], ibuf.at[b], gin[b])

    def wait(j, b):
        pltpu.make_async_copy(table_hbm.at[idx_v.at[j]], ibuf.at[b],
                              gin[b]).wait()

    def start_out(j, b):
        pltpu.async_copy(obuf.at[b],
                         scr_hbm.at[pl.ds(qrow0 + j * _QCHUNK, _QCHUNK)],
                         gout[b])

    def wait_out(j, b):
        pltpu.make_async_copy(obuf.at[b],
                              scr_hbm.at[pl.ds(qrow0 + j * _QCHUNK, _QCHUNK)],
                              gout[b]).wait()

    def pack(b):
        half = jnp.int32(0x8000)
        himask = jnp.int32(-65536)

        def row(q, carry):
            for k in range(8):
                a = ibuf[b, 2 * q, pl.ds(16 * k, 16)] + half
                c = ibuf[b, 2 * q + 1, pl.ds(16 * k, 16)] + half
                obuf[b, q, pl.ds(16 * k, 16)] = (
                    lax.shift_right_logical(a, 16) | (c & himask))
            return carry

        lax.fori_loop(0, _QCHUNK, row, 0)

    for b in range(_NBUF):
        start(b, b)

    def body(g, carry):
        for b in range(_NBUF):
            j = g * _NBUF + b
            wait(j, b)

            @pl.when(g > 0)
            def _():
                wait_out(j - _NBUF, b)

            pack(b)
            jn = j + _NBUF

            @pl.when(jn < _N_CHUNKS)
            def _():
                start(jn, b)

            start_out(j, b)
        return carry

    lax.fori_loop(0, _N_GROUPS, body, 0)

    for b in range(_NBUF):
        wait_out(_N_CHUNKS - _NBUF + b, b)


def _slab_gather(ids_slab, table_i32):
    mesh = plsc.VectorSubcoreMesh(core_axis_name="c", subcore_axis_name="s")
    return pl.kernel(
        _gather_pack_kernel,
        mesh=mesh,
        out_type=jax.ShapeDtypeStruct((_SLAB_ROWS // 2, DIM), jnp.int32),
        scratch_types=[
            pltpu.VMEM((_N_CHUNKS, _CHUNK), jnp.int32),
            pltpu.VMEM((_NBUF, _CHUNK, DIM), jnp.int32),
            pltpu.VMEM((_NBUF, _QCHUNK, DIM), jnp.int32),
        ] + [pltpu.SemaphoreType.DMA] * (2 * _NBUF),
    )(table_i32, ids_slab)


_EXP_BLK = 512
_EXP_NB = (_SLAB_ROWS // 2) // _EXP_BLK


def _expand_body(scr_ref, out_ref):
    x = scr_ref[...]
    a = lax.bitcast_convert_type(x << 16, jnp.float32)
    b = lax.bitcast_convert_type(x & jnp.int32(-65536), jnp.float32)
    out_ref[...] = jnp.stack([a, b], axis=1).reshape(2 * _EXP_BLK, DIM)


def _expand_body_aliased(x_alias_ref, scr_ref, out_ref):
    _expand_body(scr_ref, out_ref)


def _expand_first(scr):
    return pl.pallas_call(
        _expand_body,
        grid=(_EXP_NB,),
        in_specs=[pl.BlockSpec((_EXP_BLK, DIM), lambda i: (i, 0))],
        out_specs=pl.BlockSpec((2 * _EXP_BLK, DIM), lambda i: (i, 0)),
        out_shape=jax.ShapeDtypeStruct((_N_ROWS, DIM), jnp.float32),
    )(scr)


def _expand_slab(x_acc, scr, slab):
    base = slab * _EXP_NB
    return pl.pallas_call(
        _expand_body_aliased,
        grid=(_EXP_NB,),
        in_specs=[
            pl.BlockSpec(memory_space=pl.ANY),
            pl.BlockSpec((_EXP_BLK, DIM), lambda i: (i, 0)),
        ],
        out_specs=pl.BlockSpec((2 * _EXP_BLK, DIM),
                               functools.partial(lambda b, i: (b + i, 0), base)),
        out_shape=jax.ShapeDtypeStruct((_N_ROWS, DIM), jnp.float32),
        input_output_aliases={0: 0},
    )(x_acc, scr)


_POS_BLK = 16


def _pos_kernel(pos_ref, out_ref):
    out_ref[...] = jnp.broadcast_to(pos_ref[...][None], out_ref.shape)


def _pos_broadcast(W_pos_l):
    return pl.pallas_call(
        _pos_kernel,
        grid=(B // _POS_BLK,),
        in_specs=[pl.BlockSpec((L, DIM), lambda i: (0, 0))],
        out_specs=pl.BlockSpec((_POS_BLK, L, DIM), lambda i: (i, 0, 0)),
        out_shape=jax.ShapeDtypeStruct((B, L, DIM), jnp.float32),
    )(W_pos_l)


def kernel(input_ids, W_word, W_pos):
    ids_2d = input_ids.astype(jnp.int32).reshape(_N_ROWS // _CHUNK, _CHUNK)
    table_i32 = jax.lax.bitcast_convert_type(W_word, jnp.int32)
    cpw = _SLAB_ROWS // _CHUNK
    scrs = [_slab_gather(ids_2d[s * cpw:(s + 1) * cpw], table_i32)
            for s in range(_NSLAB)]
    x = _expand_first(scrs[0])
    for s in range(1, _NSLAB):
        x = _expand_slab(x, scrs[s], s)
    X_token = x.reshape(B, L, DIM)
    X_pos = _pos_broadcast(W_pos[:L])
    return (X_token, X_pos)


# speedup vs baseline: 1.1939x; 1.0009x over previous
import functools

import jax
import jax.numpy as jnp
from jax import lax
from jax.experimental import pallas as pl
from jax.experimental.pallas import tpu as pltpu
from jax.experimental.pallas import tpu_sc as plsc

VOCAB = 100000
MAX_SEQ_LEN = 512
DIM = 128
B, L = 4096, 200

_INFO = plsc.get_sparse_core_info()
_NC, _NS = _INFO.num_cores, _INFO.num_subcores
_NW = _NC * _NS

_N_ROWS = B * L
_NSLAB = 5
_SLAB_ROWS = _N_ROWS // _NSLAB
_ROWS_PER_W = _SLAB_ROWS // _NW
_CHUNK = 128
_N_CHUNKS = _ROWS_PER_W // _CHUNK
_NBUF = 5
_N_GROUPS = _N_CHUNKS // _NBUF
_QCHUNK = _CHUNK // 2


def _gather_pack_kernel(table_hbm, ids_hbm, scr_hbm, idx_v, ibuf, obuf, *sems):
    wid = lax.axis_index("s") * _NC + lax.axis_index("c")
    qrow0 = wid * (_ROWS_PER_W // 2)
    chunk0 = wid * _N_CHUNKS

    pltpu.sync_copy(ids_hbm.at[pl.ds(chunk0, _N_CHUNKS)], idx_v)

    gin = sems[:_NBUF]
    gout = sems[_NBUF:]

    def start(j, b):
        pltpu.async_copy(table_hbm.at[idx_v.at[j]], ibuf.at[b], gin[b])

    def wait(j, b):
        pltpu.make_async_copy(table_hbm.at[idx_v.at[j]], ibuf.at[b],
                              gin[b]).wait()

    def start_out(j, b):
        pltpu.async_copy(obuf.at[b],
                         scr_hbm.at[pl.ds(qrow0 + j * _QCHUNK, _QCHUNK)],
                         gout[b])

    def wait_out(j, b):
        pltpu.make_async_copy(obuf.at[b],
                              scr_hbm.at[pl.ds(qrow0 + j * _QCHUNK, _QCHUNK)],
                              gout[b]).wait()

    def pack(b):
        half = jnp.int32(0x8000)
        himask = jnp.int32(-65536)

        def row(q4, carry):
            for u in range(4):
                q = 4 * q4 + u
                for k in range(8):
                    a = ibuf[b, 2 * q, pl.ds(16 * k, 16)] + half
                    c = ibuf[b, 2 * q + 1, pl.ds(16 * k, 16)] + half
                    obuf[b, q, pl.ds(16 * k, 16)] = (
                        lax.shift_right_logical(a, 16) | (c & himask))
            return carry

        lax.fori_loop(0, _QCHUNK // 4, row, 0)

    for b in range(_NBUF):
        start(b, b)

    def body(g, carry):
        for b in range(_NBUF):
            j = g * _NBUF + b
            wait(j, b)

            @pl.when(g > 0)
            def _():
                wait_out(j - _NBUF, b)

            pack(b)
            jn = j + _NBUF

            @pl.when(jn < _N_CHUNKS)
            def _():
                start(jn, b)

            start_out(j, b)
        return carry

    lax.fori_loop(0, _N_GROUPS, body, 0)

    for b in range(_NBUF):
        wait_out(_N_CHUNKS - _NBUF + b, b)


def _slab_gather(ids_slab, table_i32):
    mesh = plsc.VectorSubcoreMesh(core_axis_name="c", subcore_axis_name="s")
    return pl.kernel(
        _gather_pack_kernel,
        mesh=mesh,
        out_type=jax.ShapeDtypeStruct((_SLAB_ROWS // 2, DIM), jnp.int32),
        scratch_types=[
            pltpu.VMEM((_N_CHUNKS, _CHUNK), jnp.int32),
            pltpu.VMEM((_NBUF, _CHUNK, DIM), jnp.int32),
            pltpu.VMEM((_NBUF, _QCHUNK, DIM), jnp.int32),
        ] + [pltpu.SemaphoreType.DMA] * (2 * _NBUF),
    )(table_i32, ids_slab)


_EXP_BLK = 512
_EXP_NB = (_SLAB_ROWS // 2) // _EXP_BLK


def _expand_body(scr_ref, out_ref):
    x = scr_ref[...]
    a = lax.bitcast_convert_type(x << 16, jnp.float32)
    b = lax.bitcast_convert_type(x & jnp.int32(-65536), jnp.float32)
    out_ref[...] = jnp.stack([a, b], axis=1).reshape(2 * _EXP_BLK, DIM)


def _expand_body_aliased(x_alias_ref, scr_ref, out_ref):
    _expand_body(scr_ref, out_ref)


def _expand_first(scr):
    return pl.pallas_call(
        _expand_body,
        grid=(_EXP_NB,),
        in_specs=[pl.BlockSpec((_EXP_BLK, DIM), lambda i: (i, 0))],
        out_specs=pl.BlockSpec((2 * _EXP_BLK, DIM), lambda i: (i, 0)),
        out_shape=jax.ShapeDtypeStruct((_N_ROWS, DIM), jnp.float32),
    )(scr)


def _expand_slab(x_acc, scr, slab):
    base = slab * _EXP_NB
    return pl.pallas_call(
        _expand_body_aliased,
        grid=(_EXP_NB,),
        in_specs=[
            pl.BlockSpec(memory_space=pl.ANY),
            pl.BlockSpec((_EXP_BLK, DIM), lambda i: (i, 0)),
        ],
        out_specs=pl.BlockSpec((2 * _EXP_BLK, DIM),
                               functools.partial(lambda b, i: (b + i, 0), base)),
        out_shape=jax.ShapeDtypeStruct((_N_ROWS, DIM), jnp.float32),
        input_output_aliases={0: 0},
    )(x_acc, scr)


_POS_BLK = 16


def _pos_kernel(pos_ref, out_ref):
    out_ref[...] = jnp.broadcast_to(pos_ref[...][None], out_ref.shape)


def _pos_broadcast(W_pos_l):
    return pl.pallas_call(
        _pos_kernel,
        grid=(B // _POS_BLK,),
        in_specs=[pl.BlockSpec((L, DIM), lambda i: (0, 0))],
        out_specs=pl.BlockSpec((_POS_BLK, L, DIM), lambda i: (i, 0, 0)),
        out_shape=jax.ShapeDtypeStruct((B, L, DIM), jnp.float32),
    )(W_pos_l)


def kernel(input_ids, W_word, W_pos):
    ids_2d = input_ids.astype(jnp.int32).reshape(_N_ROWS // _CHUNK, _CHUNK)
    table_i32 = jax.lax.bitcast_convert_type(W_word, jnp.int32)
    X_pos = _pos_broadcast(W_pos[:L])
    cpw = _SLAB_ROWS // _CHUNK

    def slab(s):
        return _slab_gather(ids_2d[s * cpw:(s + 1) * cpw], table_i32)

    scr_prev = slab(0)
    scr_next = slab(1)
    x = _expand_first(scr_prev)
    for s in range(2, _NSLAB + 1):
        scr_prev, scr_next = scr_next, (slab(s) if s < _NSLAB else None)
        x = _expand_slab(x, scr_prev, s - 1)
    X_token = x.reshape(B, L, DIM)
    return (X_token, X_pos)


# speedup vs baseline: 2.5953x; 2.1739x over previous
import jax
import jax.numpy as jnp
from jax import lax
from jax.experimental import pallas as pl
from jax.experimental.pallas import tpu as pltpu
from jax.experimental.pallas import tpu_sc as plsc

VOCAB = 100000
MAX_SEQ_LEN = 512
DIM = 128
B, L = 4096, 200

_INFO = plsc.get_sparse_core_info()
_NC, _NS = _INFO.num_cores, _INFO.num_subcores
_NW = _NC * _NS

_N_ROWS = B * L
_ROWS_PER_W = _N_ROWS // _NW
_CHUNK = 128
_N_CHUNKS = _ROWS_PER_W // _CHUNK
_NBUF = 4
_N_GROUPS = _N_CHUNKS // _NBUF


def _gather_kernel(table_hbm, ids_hbm, out_hbm, idx_v, rows_v, *sems):
    wid = lax.axis_index("s") * _NC + lax.axis_index("c")
    row0 = wid * _ROWS_PER_W
    chunk0 = wid * _N_CHUNKS

    pltpu.sync_copy(ids_hbm.at[pl.ds(chunk0, _N_CHUNKS)], idx_v)

    gin = sems[:_NBUF]
    gout = sems[_NBUF:]

    def start(j, b):
        pltpu.async_copy(table_hbm.at[idx_v.at[j]], rows_v.at[b], gin[b])

    def wait(j, b):
        pltpu.make_async_copy(table_hbm.at[idx_v.at[j]], rows_v.at[b],
                              gin[b]).wait()

    def start_out(j, b):
        pltpu.async_copy(rows_v.at[b],
                         out_hbm.at[pl.ds(row0 + j * _CHUNK, _CHUNK)],
                         gout[b])

    def wait_out(j, b):
        pltpu.make_async_copy(rows_v.at[b],
                              out_hbm.at[pl.ds(row0 + j * _CHUNK, _CHUNK)],
                              gout[b]).wait()

    for b in range(_NBUF):
        start(b, b)

    def body(g, carry):
        for b in range(_NBUF):
            j = g * _NBUF + b
            wait(j, b)
            start_out(j, b)
        for b in range(_NBUF):
            j = g * _NBUF + b
            jn = j + _NBUF
            wait_out(j, b)

            @pl.when(jn < _N_CHUNKS)
            def _():
                start(jn, b)
        return carry

    lax.fori_loop(0, _N_GROUPS, body, 0)


def _token_gather(ids_2d, W_word):
    mesh = plsc.VectorSubcoreMesh(core_axis_name="c", subcore_axis_name="s")
    return pl.kernel(
        _gather_kernel,
        mesh=mesh,
        out_type=jax.ShapeDtypeStruct((_N_ROWS, DIM), jnp.float32),
        scratch_types=[
            pltpu.VMEM((_N_CHUNKS, _CHUNK), jnp.int32),
            pltpu.VMEM((_NBUF, _CHUNK, DIM), jnp.float32),
        ] + [pltpu.SemaphoreType.DMA] * (2 * _NBUF),
    )(W_word, ids_2d)


_POS_BLK = 16


def _pos_kernel(pos_ref, out_ref):
    out_ref[...] = jnp.broadcast_to(pos_ref[...][None], out_ref.shape)


def _pos_broadcast(W_pos_l):
    return pl.pallas_call(
        _pos_kernel,
        grid=(B // _POS_BLK,),
        in_specs=[pl.BlockSpec((L, DIM), lambda i: (0, 0))],
        out_specs=pl.BlockSpec((_POS_BLK, L, DIM), lambda i: (i, 0, 0)),
        out_shape=jax.ShapeDtypeStruct((B, L, DIM), jnp.float32),
    )(W_pos_l)


def kernel(input_ids, W_word, W_pos):
    ids_2d = input_ids.astype(jnp.int32).reshape(_N_ROWS // _CHUNK, _CHUNK)
    X_token = _token_gather(ids_2d, W_word).reshape(B, L, DIM)
    X_pos = _pos_broadcast(W_pos[:L])
    return (X_token, X_pos)
